# baseline (speedup 1.0000x reference)
import functools

import jax
import jax.numpy as jnp
from jax import lax
from jax.experimental import pallas as pl
from jax.experimental.pallas import tpu as pltpu
from jax.experimental.pallas import tpu_sc as plsc

CHUNK = 256


def _make_gather(nw, nc, ch, k, d, n):
    g_rows = k * CHUNK
    rows_per_w = ch * CHUNK
    groups = ch // k
    mesh = plsc.VectorSubcoreMesh(core_axis_name="c", subcore_axis_name="s")

    @functools.partial(
        pl.kernel,
        out_type=jax.ShapeDtypeStruct((n, d), jnp.float32),
        mesh=mesh,
        scratch_types=[
            pltpu.VMEM((rows_per_w,), jnp.int32),
            pltpu.VMEM((2, g_rows, d), jnp.float32),
            pltpu.SemaphoreType.DMA,
        ],
        compiler_params=pltpu.CompilerParams(use_tc_tiling_on_sc=False),
    )
    def gather_kernel(idx_hbm, table_hbm, out_hbm, idx_v, rows_v, sem):
        wid = lax.axis_index("s") * nc + lax.axis_index("c")
        base = wid * rows_per_w
        pltpu.sync_copy(idx_hbm.at[pl.ds(base, rows_per_w)], idx_v)

        def fire_group(g, slot):
            for j in range(k):
                pltpu.async_copy(
                    table_hbm.at[idx_v.at[pl.ds((g * k + j) * CHUNK, CHUNK)]],
                    rows_v.at[slot, pl.ds(j * CHUNK, CHUNK)],
                    sem,
                )

        fire_group(0, 0)

        def body(g, carry):
            slot = lax.rem(g, 2)

            @pl.when(g + 1 < groups)
            def _():
                fire_group(g + 1, 1 - slot)

            pltpu.make_async_copy(
                out_hbm.at[pl.ds(0, g_rows)], rows_v.at[slot], sem
            ).wait()
            pltpu.sync_copy(
                rows_v.at[slot], out_hbm.at[pl.ds(base + g * g_rows, g_rows)]
            )
            return carry

        lax.fori_loop(0, groups, body, 0)

    return gather_kernel


def kernel(indices, weight):
    b, f = indices.shape
    v, d = weight.shape
    n = b * f
    info = plsc.get_sparse_core_info()
    nc, ns = info.num_cores, info.num_subcores
    nw = nc * ns
    assert n % (nw * CHUNK) == 0
    ch = n // (nw * CHUNK)
    k = 5
    while ch % k:
        k -= 1
    p = indices[0, 0] >= 0
    flat_idx = jnp.where(p, indices, 0).reshape(-1)
    table = jnp.where(p, weight, 0.0)
    out = _make_gather(nw, nc, ch, k, d, n)(flat_idx, table)
    return jnp.where(p, out.reshape(b, f, d), 0.0)


# speedup vs baseline: 1.8489x; 1.8489x over previous
import functools

import jax
import jax.numpy as jnp
from jax import lax
from jax.experimental import pallas as pl
from jax.experimental.pallas import tpu as pltpu
from jax.experimental.pallas import tpu_sc as plsc

PAIR = 2
GP = 4


def _make_gather(nw, nc, b, f, d):
    bpw = b // nw
    lp = PAIR * f
    pairs = bpw // PAIR
    groups = pairs // GP
    mesh = plsc.VectorSubcoreMesh(core_axis_name="c", subcore_axis_name="s")

    @functools.partial(
        pl.kernel,
        out_type=jax.ShapeDtypeStruct((b, f, d), jnp.float32),
        mesh=mesh,
        scratch_types=[
            pltpu.VMEM((bpw * f,), jnp.int32),
            pltpu.VMEM((2, GP, lp, d), jnp.float32),
            pltpu.SemaphoreType.DMA,
        ],
        compiler_params=pltpu.CompilerParams(use_tc_tiling_on_sc=False),
    )
    def gather_kernel(idx_hbm, table_hbm, out_hbm, idx_v, rows_v, sem):
        wid = lax.axis_index("s") * nc + lax.axis_index("c")
        b0 = wid * bpw
        pltpu.sync_copy(idx_hbm.at[wid], idx_v)

        def descs(g, slot):
            return [
                pltpu.make_async_copy(
                    table_hbm.at[idx_v.at[pl.ds((g * GP + j) * lp, lp)]],
                    rows_v.at[slot, j],
                    sem,
                )
                for j in range(GP)
            ]

        def fire(g, slot):
            for c in descs(g, slot):
                c.start()

        fire(0, 0)

        def body(g, carry):
            slot = lax.rem(g, 2)

            @pl.when(g + 1 < groups)
            def _():
                fire(g + 1, 1 - slot)

            for c in descs(g, slot):
                c.wait()
            for j in range(GP):
                for p in range(PAIR):
                    pltpu.sync_copy(
                        rows_v.at[slot, j, pl.ds(p * f, f)],
                        out_hbm.at[b0 + (g * GP + j) * PAIR + p],
                    )
            return carry

        lax.fori_loop(0, groups, body, 0)

    return gather_kernel


def kernel(indices, weight):
    b, f = indices.shape
    v, d = weight.shape
    info = plsc.get_sparse_core_info()
    nc, ns = info.num_cores, info.num_subcores
    nw = nc * ns
    assert b % (nw * PAIR * GP) == 0
    idx2 = indices.reshape(nw, (b // nw) * f)
    out = _make_gather(nw, nc, b, f, d)(idx2, weight)
    return out
